# Initial kernel scaffold; baseline (speedup 1.0000x reference)
import jax
import jax.numpy as jnp
from jax.experimental import pallas as pl


def kernel(x, edge_index, edge_attr, graph_ids, W_gcn1, b_gcn1, W_gcn2, b_gcn2, Wk1, bk1, root1, be1, Wk2, bk2, root2, be2, Wd1, bd1, Wd2, bd2, Wo, bo):
    raise NotImplementedError("write your pallas kernel here")


# speedup vs baseline: 1.9191x; 1.9191x over previous
import functools
import jax
import jax.numpy as jnp
from jax import lax
from jax.experimental import pallas as pl
from jax.experimental.pallas import tpu as pltpu

N = 10000
E = 160000
F = 128
S = 4
H = 32
G = 64
OUT = 8
NP = 10240
BLK = 400
NSTEP = N // BLK


def _t1_body(x_ref, w_ref, pdeg_ref, t1_ref, r1_ref, ac_ref):
    ac = lax.rsqrt(jnp.maximum(pdeg_ref[0] + pdeg_ref[1], 1.0))
    ac_ref[...] = ac
    a = ac[:, 0:1]
    t = jnp.dot(x_ref[...], w_ref[...], preferred_element_type=jnp.float32)
    y = a * t[:, :H]
    t1_ref[...] = jnp.concatenate([y, t[:, H:6 * H]], axis=1)
    r1_ref[...] = t[:, 6 * H:7 * H]


def _t1_call(x, wcat1, pdeg):
    return pl.pallas_call(
        _t1_body,
        grid=(NSTEP,),
        in_specs=[
            pl.BlockSpec((BLK, F), lambda i: (i, 0)),
            pl.BlockSpec((F, 7 * H), lambda i: (0, 0)),
            pl.BlockSpec((2, BLK, 16), lambda i: (0, i, 0)),
        ],
        out_specs=[
            pl.BlockSpec((BLK, 6 * H), lambda i: (i, 0)),
            pl.BlockSpec((BLK, H), lambda i: (i, 0)),
            pl.BlockSpec((BLK, 16), lambda i: (i, 0)),
        ],
        out_shape=[
            jax.ShapeDtypeStruct((N, 6 * H), jnp.float32),
            jax.ShapeDtypeStruct((N, H), jnp.float32),
            jax.ShapeDtypeStruct((N, 16), jnp.float32),
        ],
    )(x, wcat1, pdeg)


H2 = 2 * H


def _t2_body(p1_ref, ac_ref, r1_ref, w_ref, b1_ref, be1_ref, t2_ref, r2_ref):
    p = p1_ref[0] + p1_ref[1]
    ac = ac_ref[...]
    a, c = ac[:, 0:1], ac[:, 1:2]
    out1 = jnp.maximum(c * p[:, :H] + b1_ref[...], 0.0)
    out3 = jnp.maximum(p[:, H:] + r1_ref[...] + be1_ref[...], 0.0)
    o = jnp.concatenate([out1, out3], axis=1)
    t = jnp.dot(o, w_ref[...], preferred_element_type=jnp.float32)
    y2 = a * t[:, :H2]
    t2_ref[...] = jnp.concatenate([y2, t[:, H2:6 * H2]], axis=1)
    r2_ref[...] = t[:, 6 * H2:7 * H2]


def _t2_call(p1, ac, r1, wcat2, b1, be1):
    return pl.pallas_call(
        _t2_body,
        grid=(NSTEP,),
        in_specs=[
            pl.BlockSpec((2, BLK, 2 * H), lambda i: (0, i, 0)),
            pl.BlockSpec((BLK, 16), lambda i: (i, 0)),
            pl.BlockSpec((BLK, H), lambda i: (i, 0)),
            pl.BlockSpec((2 * H, 7 * H2), lambda i: (0, 0)),
            pl.BlockSpec((1, H), lambda i: (0, 0)),
            pl.BlockSpec((1, H), lambda i: (0, 0)),
        ],
        out_specs=[
            pl.BlockSpec((BLK, 6 * H2), lambda i: (i, 0)),
            pl.BlockSpec((BLK, H2), lambda i: (i, 0)),
        ],
        out_shape=[
            jax.ShapeDtypeStruct((N, 6 * H2), jnp.float32),
            jax.ShapeDtypeStruct((N, H2), jnp.float32),
        ],
    )(p1, ac, r1, wcat2, b1, be1)


def _head_body(p2_ref, ac_ref, r2_ref, gid_ref, b2_ref, be2_ref,
               wd1_ref, bd1_ref, wd2_ref, bd2_ref, wo_ref, bo_ref,
               out_ref, pool_ref):
    i = pl.program_id(0)

    @pl.when(i == 0)
    def _():
        pool_ref[...] = jnp.zeros_like(pool_ref)

    p = p2_ref[0] + p2_ref[1]
    ac = ac_ref[...]
    c = ac[:, 1:2]
    out2 = jnp.maximum(c * p[:, :H2] + b2_ref[...], 0.0)
    out4 = jnp.maximum(p[:, H2:] + r2_ref[...] + be2_ref[...], 0.0)
    o = jnp.concatenate([out2, out4], axis=1)
    gid = gid_ref[0, 0]
    rows = lax.broadcasted_iota(jnp.int32, (G, BLK), 0)
    m = (rows == gid[None, :]).astype(jnp.float32)
    pool_ref[...] += jnp.dot(m, o, preferred_element_type=jnp.float32)

    @pl.when(i == NSTEP - 1)
    def _():
        h = pool_ref[...]
        h = jnp.maximum(jnp.dot(h, wd1_ref[...],
                                preferred_element_type=jnp.float32)
                        + bd1_ref[...], 0.0)
        h = jnp.maximum(jnp.dot(h, wd2_ref[...],
                                preferred_element_type=jnp.float32)
                        + bd2_ref[...], 0.0)
        z = jnp.dot(h, wo_ref[...], preferred_element_type=jnp.float32) \
            + bo_ref[...]
        out_ref[...] = jax.nn.sigmoid(z)


def _head_call(p2, ac, r2, gid3, b2, be2, wd1, bd1, wd2, bd2, wo, bo):
    full = lambda shape: pl.BlockSpec(shape, lambda i: tuple(0 for _ in shape))
    return pl.pallas_call(
        _head_body,
        grid=(NSTEP,),
        in_specs=[
            pl.BlockSpec((2, BLK, 2 * H2), lambda i: (0, i, 0)),
            pl.BlockSpec((BLK, 16), lambda i: (i, 0)),
            pl.BlockSpec((BLK, H2), lambda i: (i, 0)),
            pl.BlockSpec((1, 1, BLK), lambda i: (i, 0, 0)),
            full((1, H2)), full((1, H2)),
            full((4 * H, H)), full((1, H)),
            full((H, H // 2)), full((1, H // 2)),
            full((H // 2, OUT)), full((1, OUT)),
        ],
        out_specs=pl.BlockSpec((G, OUT), lambda i: (0, 0)),
        out_shape=jax.ShapeDtypeStruct((G, OUT), jnp.float32),
        scratch_shapes=[pltpu.VMEM((G, 4 * H), jnp.float32)],
    )(p2, ac, r2, gid3, b2, be2, wd1, bd1, wd2, bd2, wo, bo)


def _deg_partials(src, dst):
    ones = jnp.ones((E,), jnp.float32)
    deg_out = jnp.zeros((NP,), jnp.float32).at[src].add(ones)
    deg_in = jnp.zeros((NP,), jnp.float32).at[dst].add(ones)
    p = jnp.zeros((2, NP, 16), jnp.float32)
    p = p.at[0, :, 0].set(deg_out).at[0, :, 1].set(deg_in)
    return p


def _edge_pass(table, src, dst, e, wg):
    g = table[src]
    msg_g = g[:, :wg]
    msg_e = g[:, 5 * wg:6 * wg]
    for s in range(S):
        msg_e = msg_e + e[:, s:s + 1] * g[:, (1 + s) * wg:(2 + s) * wg]
    msg = jnp.concatenate([msg_g, msg_e], axis=1)
    agg = jnp.zeros((NP, 2 * wg), jnp.float32).at[dst].add(msg)
    out = jnp.zeros((2, NP, 2 * wg), jnp.float32).at[0].set(agg)
    return out


def kernel(x, edge_index, edge_attr, graph_ids, W_gcn1, b_gcn1, W_gcn2,
           b_gcn2, Wk1, bk1, root1, be1, Wk2, bk2, root2, be2, Wd1, bd1,
           Wd2, bd2, Wo, bo):
    src, dst = edge_index[0], edge_index[1]

    wk1 = Wk1.reshape(S, F, H)
    wcat1 = jnp.concatenate(
        [W_gcn1] + [wk1[s] for s in range(S)]
        + [bk1.reshape(F, H), root1], axis=1)
    wk2 = Wk2.reshape(S, H, H2)
    right = jnp.concatenate([wk2[s] for s in range(S)]
                            + [bk2.reshape(H, H2), root2], axis=1)
    wcat2 = jnp.zeros((2 * H, 7 * H2), jnp.float32)
    wcat2 = wcat2.at[:H, :H2].set(W_gcn2).at[H:, H2:].set(right)

    gid3 = graph_ids.reshape(NSTEP, 1, BLK)

    pdeg = _deg_partials(src, dst)
    t1, r1, ac = _t1_call(x, wcat1, pdeg)
    p1 = _edge_pass(t1, src, dst, edge_attr, H)
    t2, r2 = _t2_call(p1[:, :N], ac, r1, wcat2,
                      b_gcn1.reshape(1, H), be1.reshape(1, H))
    p2 = _edge_pass(t2, src, dst, edge_attr, H2)
    out = _head_call(p2[:, :N], ac, r2, gid3,
                     b_gcn2.reshape(1, H2), be2.reshape(1, H2),
                     Wd1, bd1.reshape(1, H), Wd2, bd2.reshape(1, H // 2),
                     Wo, bo.reshape(1, OUT))
    return out


# speedup vs baseline: 2.2508x; 1.1729x over previous
import functools
import jax
import jax.numpy as jnp
from jax import lax
from jax.experimental import pallas as pl
from jax.experimental.pallas import tpu as pltpu
from jax.experimental.pallas import tpu_sc as plsc

N = 10000
E = 160000
F = 128
S = 4
H = 32
G = 64
OUT = 8
NP = 10240
BLK = 400
NSTEP = N // BLK


def _t1_body(x_ref, w_ref, pdeg_ref, t1_ref, r1_ref, ac_ref):
    dego = jnp.sum(pdeg_ref[0], axis=1, keepdims=True)
    degi = jnp.sum(pdeg_ref[1], axis=1, keepdims=True)
    a = lax.rsqrt(jnp.maximum(dego, 1.0))
    c = lax.rsqrt(jnp.maximum(degi, 1.0))
    ac = jnp.concatenate([jnp.broadcast_to(a, (BLK, 8)),
                          jnp.broadcast_to(c, (BLK, 8))], axis=1)
    ac_ref[...] = ac
    t = jnp.dot(x_ref[...], w_ref[...], preferred_element_type=jnp.float32)
    y = a * t[:, :H]
    t1_ref[...] = jnp.concatenate(
        [y, t[:, H:6 * H], jnp.zeros((BLK, 64), jnp.float32)], axis=1)
    r1_ref[...] = t[:, 6 * H:7 * H]


def _t1_call(x, wcat1, pdeg):
    return pl.pallas_call(
        _t1_body,
        grid=(NSTEP,),
        in_specs=[
            pl.BlockSpec((BLK, F), lambda i: (i, 0)),
            pl.BlockSpec((F, 7 * H), lambda i: (0, 0)),
            pl.BlockSpec((2, BLK, 16), lambda i: (0, i, 0)),
        ],
        out_specs=[
            pl.BlockSpec((BLK, 8 * H), lambda i: (i, 0)),
            pl.BlockSpec((BLK, H), lambda i: (i, 0)),
            pl.BlockSpec((BLK, 16), lambda i: (i, 0)),
        ],
        out_shape=[
            jax.ShapeDtypeStruct((N, 8 * H), jnp.float32),
            jax.ShapeDtypeStruct((N, H), jnp.float32),
            jax.ShapeDtypeStruct((N, 16), jnp.float32),
        ],
    )(x, wcat1, pdeg)


H2 = 2 * H


def _t2_body(p1_ref, ac_ref, r1_ref, w_ref, b1_ref, be1_ref, t2_ref, r2_ref):
    p = p1_ref[...]
    ac = ac_ref[...]
    a, c = ac[:, 0:1], ac[:, 8:9]
    out1 = jnp.maximum(c * p[:, :H] + b1_ref[...], 0.0)
    out3 = jnp.maximum(p[:, H:] + r1_ref[...] + be1_ref[...], 0.0)
    o = jnp.concatenate([out1, out3], axis=1)
    t = jnp.dot(o, w_ref[...], preferred_element_type=jnp.float32)
    y2 = a * t[:, :H2]
    t2_ref[...] = jnp.concatenate([y2, t[:, H2:6 * H2]], axis=1)
    r2_ref[...] = t[:, 6 * H2:7 * H2]


def _t2_call(p1, ac, r1, wcat2, b1, be1):
    return pl.pallas_call(
        _t2_body,
        grid=(NSTEP,),
        in_specs=[
            pl.BlockSpec((BLK, 2 * H), lambda i: (i, 0)),
            pl.BlockSpec((BLK, 16), lambda i: (i, 0)),
            pl.BlockSpec((BLK, H), lambda i: (i, 0)),
            pl.BlockSpec((2 * H, 7 * H2), lambda i: (0, 0)),
            pl.BlockSpec((1, H), lambda i: (0, 0)),
            pl.BlockSpec((1, H), lambda i: (0, 0)),
        ],
        out_specs=[
            pl.BlockSpec((BLK, 6 * H2), lambda i: (i, 0)),
            pl.BlockSpec((BLK, H2), lambda i: (i, 0)),
        ],
        out_shape=[
            jax.ShapeDtypeStruct((N, 6 * H2), jnp.float32),
            jax.ShapeDtypeStruct((N, H2), jnp.float32),
        ],
    )(p1, ac, r1, wcat2, b1, be1)


def _head_body(p2_ref, ac_ref, r2_ref, gid_ref, b2_ref, be2_ref,
               wd1_ref, bd1_ref, wd2_ref, bd2_ref, wo_ref, bo_ref,
               out_ref, pool_ref):
    i = pl.program_id(0)

    @pl.when(i == 0)
    def _():
        pool_ref[...] = jnp.zeros_like(pool_ref)

    p = p2_ref[...]
    ac = ac_ref[...]
    c = ac[:, 8:9]
    out2 = jnp.maximum(c * p[:, :H2] + b2_ref[...], 0.0)
    out4 = jnp.maximum(p[:, H2:] + r2_ref[...] + be2_ref[...], 0.0)
    o = jnp.concatenate([out2, out4], axis=1)
    gid = gid_ref[0, 0]
    rows = lax.broadcasted_iota(jnp.int32, (G, BLK), 0)
    m = (rows == gid[None, :]).astype(jnp.float32)
    pool_ref[...] += jnp.dot(m, o, preferred_element_type=jnp.float32,
                             precision=lax.Precision.HIGHEST)

    @pl.when(i == NSTEP - 1)
    def _():
        h = pool_ref[...]
        h = jnp.maximum(jnp.dot(h, wd1_ref[...],
                                preferred_element_type=jnp.float32)
                        + bd1_ref[...], 0.0)
        h = jnp.maximum(jnp.dot(h, wd2_ref[...],
                                preferred_element_type=jnp.float32)
                        + bd2_ref[...], 0.0)
        z = jnp.dot(h, wo_ref[...], preferred_element_type=jnp.float32) \
            + bo_ref[...]
        out_ref[...] = jax.nn.sigmoid(z)


def _head_call(p2, ac, r2, gid3, b2, be2, wd1, bd1, wd2, bd2, wo, bo):
    full = lambda shape: pl.BlockSpec(shape, lambda i: tuple(0 for _ in shape))
    return pl.pallas_call(
        _head_body,
        grid=(NSTEP,),
        in_specs=[
            pl.BlockSpec((BLK, 2 * H2), lambda i: (i, 0)),
            pl.BlockSpec((BLK, 16), lambda i: (i, 0)),
            pl.BlockSpec((BLK, H2), lambda i: (i, 0)),
            pl.BlockSpec((1, 1, BLK), lambda i: (i, 0, 0)),
            full((1, H2)), full((1, H2)),
            full((4 * H, H)), full((1, H)),
            full((H, H // 2)), full((1, H // 2)),
            full((H // 2, OUT)), full((1, OUT)),
        ],
        out_specs=pl.BlockSpec((G, OUT), lambda i: (0, 0)),
        out_shape=jax.ShapeDtypeStruct((G, OUT), jnp.float32),
        scratch_shapes=[pltpu.VMEM((G, 4 * H), jnp.float32)],
    )(p2, ac, r2, gid3, b2, be2, wd1, bd1, wd2, bd2, wo, bo)


NT = 32
NTPW = NP // NT
SB = 2000
NSB = E // SB
NCH = SB // 16

_SC_MESH = plsc.VectorSubcoreMesh(core_axis_name="c", subcore_axis_name="s")


@functools.partial(
    pl.kernel,
    out_type=jax.ShapeDtypeStruct((2, NP * 16), jnp.float32),
    mesh=_SC_MESH,
    compiler_params=pltpu.CompilerParams(needs_layout_passes=False),
    scratch_types=[
        pltpu.VMEM((SB,), jnp.int32),
        pltpu.VMEM((NTPW * 16 + 16,), jnp.float32),
        pltpu.VMEM((NTPW * 16 + 16,), jnp.float32),
    ],
)
def _deg_kernel(src_hbm, dst_hbm, out_hbm, ibuf, ho, hi):
    cid = lax.axis_index("c")
    sid = lax.axis_index("s")
    tid = sid * 2 + cid
    lo = tid * NTPW
    lanes = lax.iota(jnp.int32, 16)
    ones = jnp.ones((16,), jnp.float32)
    zv = jnp.zeros((16,), jnp.float32)

    @pl.loop(0, NTPW + 1)
    def _(i):
        ho[pl.ds(i * 16, 16)] = zv
        hi[pl.ds(i * 16, 16)] = zv

    for hist, arr in ((ho, src_hbm), (hi, dst_hbm)):
        @pl.loop(0, NSB)
        def _(b):
            pltpu.sync_copy(arr.at[pl.ds(b * SB, SB)], ibuf)

            @pl.loop(0, NCH)
            def _(ch):
                v = ibuf[pl.ds(ch * 16, 16)] - lo
                m = (v >= 0) & (v < NTPW)
                addr = jnp.where(m, v * 16, NTPW * 16) + lanes
                plsc.addupdate_scatter(hist, [addr], ones)

    pltpu.sync_copy(ho.at[pl.ds(0, NTPW * 16)],
                    out_hbm.at[0, pl.ds(lo * 16, NTPW * 16)])
    pltpu.sync_copy(hi.at[pl.ds(0, NTPW * 16)],
                    out_hbm.at[1, pl.ds(lo * 16, NTPW * 16)])


def _lane_take(vec, lv):
    return lax.gather(
        vec, lv[:, None],
        lax.GatherDimensionNumbers(offset_dims=(), collapsed_slice_dims=(0,),
                                   start_index_map=(0,)),
        (1,), mode=lax.GatherScatterMode.PROMISE_IN_BOUNDS)


def _make_edge_sc(wg):
    wt = ((6 * wg + 127) // 128) * 128
    wm = 2 * wg

    @functools.partial(
        pl.kernel,
        out_type=jax.ShapeDtypeStruct((NP * wm,), jnp.float32),
        mesh=_SC_MESH,
        compiler_params=pltpu.CompilerParams(needs_layout_passes=False),
        scratch_types=[
            pltpu.VMEM((SB,), jnp.int32),
            pltpu.VMEM((SB,), jnp.int32),
            pltpu.VMEM((4 * SB,), jnp.float32),
            pltpu.VMEM((256,), jnp.int32),
            pltpu.VMEM((256,), jnp.int32),
            pltpu.VMEM((4 * 256,), jnp.float32),
            pltpu.VMEM((128, wt), jnp.float32),
            pltpu.VMEM(((NTPW + 1) * wm,), jnp.float32),
            pltpu.SemaphoreType.DMA,
        ],
    )
    def _edge_kernel(table_hbm, src_hbm, dst_hbm, attr_hbm, out_hbm,
                     dbuf, sbuf, abuf, csrc, cdst, cattr, rows, acc, sem):
        cid = lax.axis_index("c")
        sid = lax.axis_index("s")
        tid = sid * 2 + cid
        lo = tid * NTPW
        lanes = lax.iota(jnp.int32, 16)
        zv = jnp.zeros((16,), jnp.float32)

        @pl.loop(0, (NTPW + 1) * wm // 16)
        def _(i):
            acc[pl.ds(i * 16, 16)] = zv

        def drain():
            pltpu.async_copy(table_hbm.at[csrc.at[pl.ds(0, 128)]], rows,
                             sem).wait()

            @pl.loop(0, 128)
            def _(i):
                gbase = (i // 16) * 16
                lane = i - gbase
                lv = jnp.full((16,), lane, jnp.int32)
                dlv = _lane_take(cdst[pl.ds(gbase, 16)], lv)
                evs = [_lane_take(cattr[pl.ds(s * 256 + gbase, 16)], lv)
                       for s in range(S)]
                for c in range(wg // 16):
                    v = rows[i, pl.ds(5 * wg + c * 16, 16)]
                    for s in range(S):
                        v = v + evs[s] * rows[i, pl.ds((1 + s) * wg + c * 16, 16)]
                    plsc.addupdate_scatter(
                        acc, [dlv * wm + (lanes + (wg + c * 16))], v)
                for c in range(wg // 16):
                    plsc.addupdate_scatter(acc, [dlv * wm + (lanes + c * 16)],
                                           rows[i, pl.ds(c * 16, 16)])

            csrc[pl.ds(0, 16)] = csrc[pl.ds(128, 16)]
            cdst[pl.ds(0, 16)] = cdst[pl.ds(128, 16)]
            for s in range(S):
                cattr[pl.ds(s * 256, 16)] = cattr[pl.ds(s * 256 + 128, 16)]

        def _scan_block(b, off):
            base = b * SB
            pltpu.sync_copy(dst_hbm.at[pl.ds(base, SB)], dbuf)
            pltpu.sync_copy(src_hbm.at[pl.ds(base, SB)], sbuf)
            for s in range(S):
                pltpu.sync_copy(attr_hbm.at[pl.ds(s * E + base, SB)],
                                abuf.at[pl.ds(s * SB, SB)])

            @pl.loop(0, NCH, init_carry=off)
            def off2(ch, off):
                dl = dbuf[pl.ds(ch * 16, 16)] - lo
                m = (dl >= 0) & (dl < NTPW)
                plsc.store_compressed(csrc.at[pl.ds(off, 16)],
                                      sbuf[pl.ds(ch * 16, 16)], mask=m)
                plsc.store_compressed(cdst.at[pl.ds(off, 16)], dl, mask=m)
                for s in range(S):
                    plsc.store_compressed(cattr.at[pl.ds(s * 256 + off, 16)],
                                          abuf[pl.ds(s * SB + ch * 16, 16)],
                                          mask=m)
                off = off + jnp.sum(jnp.where(m, 1, 0).astype(jnp.int32))
                pl.when(off >= 128)(drain)
                return jnp.where(off >= 128, off - 128, off)

            return off2

        off = pl.loop(0, NSB, init_carry=jnp.int32(0))(_scan_block)

        @pl.when(off > 0)
        def _():
            for ch in range(8):
                pos = lanes + ch * 16
                keep = pos < off
                csrc[pl.ds(ch * 16, 16)] = jnp.where(
                    keep, csrc[pl.ds(ch * 16, 16)], 0)
                cdst[pl.ds(ch * 16, 16)] = jnp.where(
                    keep, cdst[pl.ds(ch * 16, 16)], NTPW)
                for s in range(S):
                    cattr[pl.ds(s * 256 + ch * 16, 16)] = jnp.where(
                        keep, cattr[pl.ds(s * 256 + ch * 16, 16)], 0.0)
            drain()

        pltpu.sync_copy(acc.at[pl.ds(0, NTPW * wm)],
                        out_hbm.at[pl.ds(lo * wm, NTPW * wm)])

    return _edge_kernel


_edge_sc_1 = _make_edge_sc(H)
_edge_sc_2 = _make_edge_sc(H2)


def kernel(x, edge_index, edge_attr, graph_ids, W_gcn1, b_gcn1, W_gcn2,
           b_gcn2, Wk1, bk1, root1, be1, Wk2, bk2, root2, be2, Wd1, bd1,
           Wd2, bd2, Wo, bo):
    src, dst = edge_index[0], edge_index[1]

    wk1 = Wk1.reshape(S, F, H)
    wcat1 = jnp.concatenate(
        [W_gcn1] + [wk1[s] for s in range(S)]
        + [bk1.reshape(F, H), root1], axis=1)
    wk2 = Wk2.reshape(S, H, H2)
    right = jnp.concatenate([wk2[s] for s in range(S)]
                            + [bk2.reshape(H, H2), root2], axis=1)
    wcat2 = jnp.zeros((2 * H, 7 * H2), jnp.float32)
    wcat2 = wcat2.at[:H, :H2].set(W_gcn2).at[H:, H2:].set(right)

    gid3 = graph_ids.reshape(NSTEP, 1, BLK)
    attr_t = edge_attr.T.reshape(-1)

    pdeg = _deg_kernel(src, dst).reshape(2, NP, 16)
    t1, r1, ac = _t1_call(x, wcat1, pdeg)
    p1 = _edge_sc_1(t1, src, dst, attr_t).reshape(NP, 2 * H)
    t2, r2 = _t2_call(p1[:N], ac, r1, wcat2,
                      b_gcn1.reshape(1, H), be1.reshape(1, H))
    p2 = _edge_sc_2(t2, src, dst, attr_t).reshape(NP, 2 * H2)
    out = _head_call(p2[:N], ac, r2, gid3,
                     b_gcn2.reshape(1, H2), be2.reshape(1, H2),
                     Wd1, bd1.reshape(1, H), Wd2, bd2.reshape(1, H // 2),
                     Wo, bo.reshape(1, OUT))
    return out
